# Initial kernel scaffold; baseline (speedup 1.0000x reference)
import functools

import jax
import jax.numpy as jnp
from jax import lax
from jax.experimental import pallas as pl
from jax.experimental.pallas import tpu as pltpu
from jax.experimental.pallas import tpu_sc as plsc

N = 10000
E = 320000
D = 128
NC = 2
NS = 16
CH = 128
NCHUNK = E // CH
CPT = NCHUNK // NS
EXTRA = NCHUNK - CPT * NS
IB = 12
NB_IDX = CPT // IB
RPT = N // NS


def _leaky(x):
    return jnp.where(x >= 0, x, 0.01 * x)


def _proj_body(f_ref, w_ref, b_ref, o_ref):
    acc = jnp.dot(f_ref[0], w_ref[0], preferred_element_type=jnp.float32)
    acc = acc + b_ref[0]
    o_ref[0] = _leaky(acc)


def _tc_project(feats, Ws, bs):
    BR = 1000
    grid = (2, N // BR)
    return pl.pallas_call(
        _proj_body,
        grid=grid,
        in_specs=[
            pl.BlockSpec((1, BR, D), lambda e, i: (e, i, 0)),
            pl.BlockSpec((1, D, D), lambda e, i: (e, 0, 0)),
            pl.BlockSpec((1, 1, D), lambda e, i: (e, 0, 0)),
        ],
        out_specs=pl.BlockSpec((1, BR, D), lambda e, i: (e, i, 0)),
        out_shape=jax.ShapeDtypeStruct((2, N, D), jnp.float32),
    )(feats, Ws, bs)


def _sc_agg_body(wh_hbm, src_hbm, dst_hbm, zf_hbm, zc_hbm,
                 sums_hbm, cnts_hbm,
                 src_v, dst_v, src_t, dst_t, rows_v, ones_v,
                 accum_sh, cnt_sh, gsem):
    c = lax.axis_index("c")
    s = lax.axis_index("s")

    e0 = jnp.where(lax.iota(jnp.int32, 16) == 0, 1.0, 0.0).astype(jnp.float32)

    def _init_ones(i, _):
        ones_v[i, :] = e0
        return 0
    lax.fori_loop(0, CH, _init_ones, 0)

    row0 = s * RPT
    pltpu.sync_copy(zf_hbm.at[pl.ds(row0, RPT)], accum_sh.at[pl.ds(row0, RPT)])
    pltpu.sync_copy(zc_hbm.at[pl.ds(row0, RPT)], cnt_sh.at[pl.ds(row0, RPT)])
    plsc.subcore_barrier()

    chunk0 = s * CPT

    def _one_chunk(sv, dv):
        pltpu.async_copy(wh_hbm.at[sv], rows_v, gsem).wait()
        pltpu.sync_copy(rows_v, accum_sh.at[dv], add=True)
        pltpu.sync_copy(ones_v, cnt_sh.at[dv], add=True)

    def _block(t, _):
        base = chunk0 + t * IB
        pltpu.sync_copy(src_hbm.at[c, pl.ds(base, IB)], src_v)
        pltpu.sync_copy(dst_hbm.at[c, pl.ds(base, IB)], dst_v)
        for j in range(IB):
            _one_chunk(src_v.at[j], dst_v.at[j])
        return 0

    lax.fori_loop(0, NB_IDX, _block, 0)

    @pl.when(s < EXTRA)
    def _():
        base = CPT * NS + s
        pltpu.sync_copy(src_hbm.at[c, pl.ds(base, 1)], src_t)
        pltpu.sync_copy(dst_hbm.at[c, pl.ds(base, 1)], dst_t)
        _one_chunk(src_t.at[0], dst_t.at[0])

    plsc.subcore_barrier()

    pltpu.sync_copy(accum_sh.at[pl.ds(row0, RPT)],
                    sums_hbm.at[c, pl.ds(row0, RPT)])
    pltpu.sync_copy(cnt_sh.at[pl.ds(row0, RPT)],
                    cnts_hbm.at[c, pl.ds(row0, RPT)])


def _sc_aggregate(wh_flat, src, dst, zf, zc):
    mesh = plsc.VectorSubcoreMesh(core_axis_name="c", subcore_axis_name="s")
    k = pl.kernel(
        _sc_agg_body,
        out_type=[
            jax.ShapeDtypeStruct((2, N, D), jnp.float32),
            jax.ShapeDtypeStruct((2, N, 16), jnp.float32),
        ],
        mesh=mesh,
        scratch_types=[
            pltpu.VMEM((IB, CH), jnp.int32),
            pltpu.VMEM((IB, CH), jnp.int32),
            pltpu.VMEM((1, CH), jnp.int32),
            pltpu.VMEM((1, CH), jnp.int32),
            pltpu.VMEM((CH, D), jnp.float32),
            pltpu.VMEM((CH, 16), jnp.float32),
            pltpu.VMEM_SHARED((N, D), jnp.float32),
            pltpu.VMEM_SHARED((N, 16), jnp.float32),
            pltpu.SemaphoreType.DMA,
        ],
    )
    return k(wh_flat, src, dst, zf, zc)


def _final_body(s_ref, c_ref, f_ref, w_ref, b_ref, o_ref):
    cnt = c_ref[0][:, 0:1]
    h = s_ref[0] / jnp.maximum(cnt, 1.0)
    acc = jnp.dot(h, w_ref[...], preferred_element_type=jnp.float32)
    acc = acc + b_ref[...]
    o_ref[0] = _leaky(acc) + f_ref[0]


def _tc_final(sums, cnts, feats, W_h, b_h):
    BR = 1000
    grid = (2, N // BR)
    return pl.pallas_call(
        _final_body,
        grid=grid,
        in_specs=[
            pl.BlockSpec((1, BR, D), lambda e, i: (e, i, 0)),
            pl.BlockSpec((1, BR, 16), lambda e, i: (e, i, 0)),
            pl.BlockSpec((1, BR, D), lambda e, i: (e, i, 0)),
            pl.BlockSpec((D, D), lambda e, i: (0, 0)),
            pl.BlockSpec((1, D), lambda e, i: (0, 0)),
        ],
        out_specs=pl.BlockSpec((1, BR, D), lambda e, i: (e, i, 0)),
        out_shape=jax.ShapeDtypeStruct((2, N, D), jnp.float32),
    )(sums, cnts, feats, W_h, b_h)


def kernel(feat_table, feat_column, edge_t2c, edge_c2t,
           W_t2c, b_t2c, W_c2t, b_c2t, W_h, b_h):
    feats01 = jnp.stack([feat_table, feat_column])
    Ws = jnp.stack([W_t2c, W_c2t])
    bs = jnp.stack([b_t2c, b_c2t]).reshape(2, 1, D)
    wh = _tc_project(feats01, Ws, bs)
    wh_flat = wh.reshape(2 * N, D)

    src = jnp.stack([edge_t2c[0].astype(jnp.int32),
                     edge_c2t[0].astype(jnp.int32) + N]).reshape(2, NCHUNK, CH)
    dst = jnp.stack([edge_t2c[1].astype(jnp.int32),
                     edge_c2t[1].astype(jnp.int32)]).reshape(2, NCHUNK, CH)

    zf = jnp.zeros((N, D), jnp.float32)
    zc = jnp.zeros((N, 16), jnp.float32)
    sums, cnts = _sc_aggregate(wh_flat, src, dst, zf, zc)

    feats_rev = jnp.stack([feat_column, feat_table])
    out = _tc_final(sums, cnts, feats_rev, W_h, b_h.reshape(1, D))
    return out[1], out[0]


# speedup vs baseline: 7.5588x; 7.5588x over previous
import functools

import jax
import jax.numpy as jnp
from jax import lax
from jax.experimental import pallas as pl
from jax.experimental.pallas import tpu as pltpu
from jax.experimental.pallas import tpu_sc as plsc

N = 10000
E = 320000
D = 128
NC = 2
NS = 16
CH = 128
NCHUNK = E // CH
CPT = NCHUNK // NS
EXTRA = NCHUNK - CPT * NS
IB = 12
NB_IDX = CPT // IB
RPT = N // NS


def _leaky(x):
    return jnp.where(x >= 0, x, 0.01 * x)


def _proj_body(f_ref, w_ref, b_ref, o_ref):
    acc = jnp.dot(f_ref[0], w_ref[0], preferred_element_type=jnp.float32)
    acc = acc + b_ref[0]
    o_ref[0] = _leaky(acc)


def _tc_project(feats, Ws, bs):
    BR = 1000
    grid = (2, N // BR)
    return pl.pallas_call(
        _proj_body,
        grid=grid,
        in_specs=[
            pl.BlockSpec((1, BR, D), lambda e, i: (e, i, 0)),
            pl.BlockSpec((1, D, D), lambda e, i: (e, 0, 0)),
            pl.BlockSpec((1, 1, D), lambda e, i: (e, 0, 0)),
        ],
        out_specs=pl.BlockSpec((1, BR, D), lambda e, i: (e, i, 0)),
        out_shape=jax.ShapeDtypeStruct((2, N, D), jnp.float32),
    )(feats, Ws, bs)


def _sc_agg_body(wh_hbm, src_hbm, dst_hbm, zf_hbm, zc_hbm,
                 sums_hbm, cnts_hbm,
                 src_v, dst_v, src_t, dst_t, rows_v, ones_v,
                 accum_sh, cnt_sh, gsem):
    c = lax.axis_index("c")
    s = lax.axis_index("s")

    e0 = jnp.where(lax.iota(jnp.int32, 16) == 0, 1.0, 0.0).astype(jnp.float32)

    def _init_ones(i, _):
        ones_v[i, :] = e0
        return 0
    lax.fori_loop(0, CH, _init_ones, 0)

    row0 = s * RPT
    pltpu.sync_copy(zf_hbm.at[pl.ds(row0, RPT)], accum_sh.at[pl.ds(row0, RPT)])
    pltpu.sync_copy(zc_hbm.at[pl.ds(row0, RPT)], cnt_sh.at[pl.ds(row0, RPT)])
    plsc.subcore_barrier()

    chunk0 = s * CPT

    def _one_chunk(sv, dv):
        pltpu.async_copy(wh_hbm.at[sv], rows_v, gsem).wait()
        pltpu.sync_copy(rows_v, accum_sh.at[dv], add=True)
        pltpu.sync_copy(ones_v, cnt_sh.at[dv], add=True)

    def _block(t, _):
        base = chunk0 + t * IB
        pltpu.sync_copy(src_hbm.at[c, pl.ds(base, IB)], src_v)
        pltpu.sync_copy(dst_hbm.at[c, pl.ds(base, IB)], dst_v)
        for j in range(IB):
            _one_chunk(src_v.at[j], dst_v.at[j])
        return 0

    lax.fori_loop(0, NB_IDX, _block, 0)

    @pl.when(s < EXTRA)
    def _():
        base = CPT * NS + s
        pltpu.sync_copy(src_hbm.at[c, pl.ds(base, 1)], src_t)
        pltpu.sync_copy(dst_hbm.at[c, pl.ds(base, 1)], dst_t)
        _one_chunk(src_t.at[0], dst_t.at[0])

    plsc.subcore_barrier()

    pltpu.sync_copy(accum_sh.at[pl.ds(row0, RPT)],
                    sums_hbm.at[c, pl.ds(row0, RPT)])
    pltpu.sync_copy(cnt_sh.at[pl.ds(row0, RPT)],
                    cnts_hbm.at[c, pl.ds(row0, RPT)])


def _sc_aggregate(wh_flat, src, dst, zf, zc):
    mesh = plsc.VectorSubcoreMesh(core_axis_name="c", subcore_axis_name="s")
    k = pl.kernel(
        _sc_agg_body,
        out_type=[
            jax.ShapeDtypeStruct((2, N, D), jnp.float32),
            jax.ShapeDtypeStruct((2, N, 16), jnp.float32),
        ],
        mesh=mesh,
        scratch_types=[
            pltpu.VMEM((IB, CH), jnp.int32),
            pltpu.VMEM((IB, CH), jnp.int32),
            pltpu.VMEM((1, CH), jnp.int32),
            pltpu.VMEM((1, CH), jnp.int32),
            pltpu.VMEM((CH, D), jnp.float32),
            pltpu.VMEM((CH, 16), jnp.float32),
            pltpu.VMEM_SHARED((N, D), jnp.float32),
            pltpu.VMEM_SHARED((N, 16), jnp.float32),
            pltpu.SemaphoreType.DMA,
        ],
        compiler_params=pltpu.CompilerParams(use_tc_tiling_on_sc=False),
    )
    return k(wh_flat, src, dst, zf, zc)


def _final_body(s_ref, c_ref, f_ref, w_ref, b_ref, o_ref):
    cnt = c_ref[0][:, 0:1]
    h = s_ref[0] / jnp.maximum(cnt, 1.0)
    acc = jnp.dot(h, w_ref[...], preferred_element_type=jnp.float32)
    acc = acc + b_ref[...]
    o_ref[0] = _leaky(acc) + f_ref[0]


def _tc_final(sums, cnts, feats, W_h, b_h):
    BR = 1000
    grid = (2, N // BR)
    return pl.pallas_call(
        _final_body,
        grid=grid,
        in_specs=[
            pl.BlockSpec((1, BR, D), lambda e, i: (e, i, 0)),
            pl.BlockSpec((1, BR, 16), lambda e, i: (e, i, 0)),
            pl.BlockSpec((1, BR, D), lambda e, i: (e, i, 0)),
            pl.BlockSpec((D, D), lambda e, i: (0, 0)),
            pl.BlockSpec((1, D), lambda e, i: (0, 0)),
        ],
        out_specs=pl.BlockSpec((1, BR, D), lambda e, i: (e, i, 0)),
        out_shape=jax.ShapeDtypeStruct((2, N, D), jnp.float32),
    )(sums, cnts, feats, W_h, b_h)


def kernel(feat_table, feat_column, edge_t2c, edge_c2t,
           W_t2c, b_t2c, W_c2t, b_c2t, W_h, b_h):
    feats01 = jnp.stack([feat_table, feat_column])
    Ws = jnp.stack([W_t2c, W_c2t])
    bs = jnp.stack([b_t2c, b_c2t]).reshape(2, 1, D)
    wh = _tc_project(feats01, Ws, bs)
    wh_flat = wh.reshape(2 * N, D)

    src = jnp.stack([edge_t2c[0].astype(jnp.int32),
                     edge_c2t[0].astype(jnp.int32) + N]).reshape(2, NCHUNK, CH)
    dst = jnp.stack([edge_t2c[1].astype(jnp.int32),
                     edge_c2t[1].astype(jnp.int32)]).reshape(2, NCHUNK, CH)

    zf = jnp.zeros((N, D), jnp.float32)
    zc = jnp.zeros((N, 16), jnp.float32)
    sums, cnts = _sc_aggregate(wh_flat, src, dst, zf, zc)

    feats_rev = jnp.stack([feat_column, feat_table])
    out = _tc_final(sums, cnts, feats_rev, W_h, b_h.reshape(1, D))
    return out[1], out[0]


# speedup vs baseline: 9.9253x; 1.3131x over previous
import jax
import jax.numpy as jnp
from jax import lax
from jax.experimental import pallas as pl
from jax.experimental.pallas import tpu as pltpu
from jax.experimental.pallas import tpu_sc as plsc

N = 10000
E = 320000
D = 128
DW = 144
NS = 16
CH = 128
NCHUNK = E // CH
CPT = NCHUNK // NS
EXTRA = NCHUNK - CPT * NS
IB = 6
NBLK = CPT // IB
RPT = N // NS


def _leaky(x):
    return jnp.where(x >= 0, x, 0.01 * x)


def _proj_body(f_ref, w_ref, b_ref, o_ref):
    acc = jnp.dot(f_ref[0], w_ref[0], preferred_element_type=jnp.float32)
    acc = acc + b_ref[0]
    o_ref[0, :, 0:D] = _leaky(acc)
    lane = lax.broadcasted_iota(jnp.int32, (acc.shape[0], DW - D), 1)
    o_ref[0, :, D:DW] = jnp.where(lane == 0, 1.0, 0.0)


def _tc_project(feats, Ws, bs):
    BR = 1000
    grid = (2, N // BR)
    return pl.pallas_call(
        _proj_body,
        grid=grid,
        in_specs=[
            pl.BlockSpec((1, BR, D), lambda e, i: (e, i, 0)),
            pl.BlockSpec((1, D, D), lambda e, i: (e, 0, 0)),
            pl.BlockSpec((1, 1, D), lambda e, i: (e, 0, 0)),
        ],
        out_specs=pl.BlockSpec((1, BR, DW), lambda e, i: (e, i, 0)),
        out_shape=jax.ShapeDtypeStruct((2, N, DW), jnp.float32),
    )(feats, Ws, bs)


def _sc_agg_body(wh_hbm, src_hbm, dst_hbm, zf_hbm, sums_hbm,
                 sidx0, sidx1, didx0, didx1,
                 rows0, rows1,
                 accum_sh,
                 g0, g1, s0, s1, isem):
    c = lax.axis_index("c")
    s = lax.axis_index("s")
    rows = (rows0, rows1)
    gsem = (g0, g1)
    ssem = (s0, s1)
    sidx = (sidx0, sidx1)
    didx = (didx0, didx1)

    row0 = s * RPT
    pltpu.sync_copy(zf_hbm.at[pl.ds(row0, RPT)], accum_sh.at[pl.ds(row0, RPT)])
    plsc.subcore_barrier()

    chunk0 = s * CPT

    def _gather(iref, k):
        pltpu.async_copy(wh_hbm.at[iref], rows[k], gsem[k])

    def _wait_gather(k):
        pltpu.make_async_copy(wh_hbm.at[sidx0.at[0]], rows[k], gsem[k]).wait()

    def _scatter(dref, k):
        pltpu.async_copy(rows[k], accum_sh.at[dref], ssem[k], add=True)

    def _wait_scatter(k):
        pltpu.make_async_copy(rows[k], accum_sh.at[didx0.at[0]],
                              ssem[k]).wait()

    def _wait_idx(bb):
        pltpu.make_async_copy(src_hbm.at[c, pl.ds(chunk0, IB)], sidx[bb],
                              isem).wait()
        pltpu.make_async_copy(dst_hbm.at[c, pl.ds(chunk0, IB)], didx[bb],
                              isem).wait()

    pltpu.sync_copy(src_hbm.at[c, pl.ds(chunk0, IB)], sidx0)
    pltpu.sync_copy(dst_hbm.at[c, pl.ds(chunk0, IB)], didx0)
    _gather(sidx0.at[0], 0)
    _gather(sidx0.at[1], 1)

    def _do_block(t, bb):
        @pl.when(t + 1 < NBLK)
        def _():
            nb = chunk0 + (t + 1) * IB
            pltpu.async_copy(src_hbm.at[c, pl.ds(nb, IB)], sidx[1 - bb], isem)
            pltpu.async_copy(dst_hbm.at[c, pl.ds(nb, IB)], didx[1 - bb], isem)

        for jl in range(IB):
            k = jl % 2
            _wait_gather(k)
            _scatter(didx[bb].at[jl], k)
            _wait_scatter(k)
            nxt = jl + 2
            if nxt < IB:
                _gather(sidx[bb].at[nxt], k)
            else:
                if jl == IB - 2:
                    @pl.when(t + 1 < NBLK)
                    def _():
                        _wait_idx(1 - bb)

                @pl.when(t + 1 < NBLK)
                def _():
                    _gather(sidx[1 - bb].at[nxt - IB], k)

    def _pair(u, _):
        _do_block(2 * u, 0)
        _do_block(2 * u + 1, 1)
        return 0

    lax.fori_loop(0, NBLK // 2, _pair, 0)

    @pl.when(s < EXTRA)
    def _():
        base = CPT * NS + s
        pltpu.sync_copy(src_hbm.at[c, pl.ds(base, 1)], sidx0.at[pl.ds(0, 1)])
        pltpu.sync_copy(dst_hbm.at[c, pl.ds(base, 1)], didx0.at[pl.ds(0, 1)])
        pltpu.async_copy(wh_hbm.at[sidx0.at[0]], rows0, g0).wait()
        pltpu.sync_copy(rows0, accum_sh.at[didx0.at[0]], add=True)

    plsc.subcore_barrier()

    pltpu.sync_copy(accum_sh.at[pl.ds(row0, RPT)],
                    sums_hbm.at[c, pl.ds(row0, RPT)])


def _sc_aggregate(wh_flat, src, dst, zf):
    mesh = plsc.VectorSubcoreMesh(core_axis_name="c", subcore_axis_name="s")
    k = pl.kernel(
        _sc_agg_body,
        out_type=jax.ShapeDtypeStruct((2, N, DW), jnp.float32),
        mesh=mesh,
        scratch_types=[
            pltpu.VMEM((IB, CH), jnp.int32),
            pltpu.VMEM((IB, CH), jnp.int32),
            pltpu.VMEM((IB, CH), jnp.int32),
            pltpu.VMEM((IB, CH), jnp.int32),
            pltpu.VMEM((CH, DW), jnp.float32),
            pltpu.VMEM((CH, DW), jnp.float32),
            pltpu.VMEM_SHARED((N, DW), jnp.float32),
            pltpu.SemaphoreType.DMA,
            pltpu.SemaphoreType.DMA,
            pltpu.SemaphoreType.DMA,
            pltpu.SemaphoreType.DMA,
            pltpu.SemaphoreType.DMA,
        ],
        compiler_params=pltpu.CompilerParams(use_tc_tiling_on_sc=False),
    )
    return k(wh_flat, src, dst, zf)


def _final_body(s_ref, f_ref, w_ref, b_ref, o_ref):
    cnt = s_ref[0][:, D:D + 1]
    h = s_ref[0][:, 0:D] / jnp.maximum(cnt, 1.0)
    acc = jnp.dot(h, w_ref[...], preferred_element_type=jnp.float32)
    acc = acc + b_ref[...]
    o_ref[0] = _leaky(acc) + f_ref[0]


def _tc_final(sums, feats, W_h, b_h):
    BR = 1000
    grid = (2, N // BR)
    return pl.pallas_call(
        _final_body,
        grid=grid,
        in_specs=[
            pl.BlockSpec((1, BR, DW), lambda e, i: (e, i, 0)),
            pl.BlockSpec((1, BR, D), lambda e, i: (e, i, 0)),
            pl.BlockSpec((D, D), lambda e, i: (0, 0)),
            pl.BlockSpec((1, D), lambda e, i: (0, 0)),
        ],
        out_specs=pl.BlockSpec((1, BR, D), lambda e, i: (e, i, 0)),
        out_shape=jax.ShapeDtypeStruct((2, N, D), jnp.float32),
    )(sums, feats, W_h, b_h)


def kernel(feat_table, feat_column, edge_t2c, edge_c2t,
           W_t2c, b_t2c, W_c2t, b_c2t, W_h, b_h):
    feats01 = jnp.stack([feat_table, feat_column])
    Ws = jnp.stack([W_t2c, W_c2t])
    bs = jnp.stack([b_t2c, b_c2t]).reshape(2, 1, D)
    wh = _tc_project(feats01, Ws, bs)
    wh_flat = wh.reshape(2 * N, DW)

    src = jnp.stack([edge_t2c[0].astype(jnp.int32),
                     edge_c2t[0].astype(jnp.int32) + N]).reshape(2, NCHUNK, CH)
    dst = jnp.stack([edge_t2c[1].astype(jnp.int32),
                     edge_c2t[1].astype(jnp.int32)]).reshape(2, NCHUNK, CH)

    zf = jnp.zeros((N, DW), jnp.float32)
    sums = _sc_aggregate(wh_flat, src, dst, zf)

    feats_rev = jnp.stack([feat_column, feat_table])
    out = _tc_final(sums, feats_rev, W_h, b_h.reshape(1, D))
    return out[1], out[0]


# speedup vs baseline: 12.8188x; 1.2915x over previous
import jax
import jax.numpy as jnp
from jax import lax
from jax.experimental import pallas as pl
from jax.experimental.pallas import tpu as pltpu
from jax.experimental.pallas import tpu_sc as plsc

N = 10000
E = 320000
D = 128
NS = 16
CH = 128
NCHUNK = E // CH
CPT = NCHUNK // NS
EXTRA = NCHUNK - CPT * NS
IB = 6
NBLK = CPT // IB
RPT = N // NS
NPAD = 640


def _leaky(x):
    return jnp.where(x >= 0, x, 0.01 * x)


def _proj_body(ft_ref, fc_ref, w_ref, b_ref, o_ref):
    e = pl.program_id(0)
    x = jnp.where(e == 0, ft_ref[...], fc_ref[...])
    acc = jnp.dot(x, w_ref[0], preferred_element_type=jnp.float32)
    o_ref[0] = _leaky(acc + b_ref[0])


def _tc_project(feat_table, feat_column, Ws, bs):
    BR = 1000
    grid = (2, N // BR)
    return pl.pallas_call(
        _proj_body,
        grid=grid,
        in_specs=[
            pl.BlockSpec((BR, D), lambda e, i: (i, 0)),
            pl.BlockSpec((BR, D), lambda e, i: (i, 0)),
            pl.BlockSpec((1, D, D), lambda e, i: (e, 0, 0)),
            pl.BlockSpec((1, 1, D), lambda e, i: (e, 0, 0)),
        ],
        out_specs=pl.BlockSpec((1, BR, D), lambda e, i: (e, i, 0)),
        out_shape=jax.ShapeDtypeStruct((2, N, D), jnp.float32),
    )(feat_table, feat_column, Ws, bs)


def _sc_agg_body(wh_hbm, e_hbm, sums_hbm, cnts_hbm,
                 sidx0, sidx1, didx0, didx1,
                 rows0, rows1, cnt_local,
                 accum_sh, cntg_sh,
                 g0, g1, s0, s1, isem, csem):
    c = lax.axis_index("c")
    s = lax.axis_index("s")
    rows = (rows0, rows1)
    gsem = (g0, g1)
    ssem = (s0, s1)
    sidx = (sidx0, sidx1)
    didx = (didx0, didx1)

    zf32 = jnp.zeros((16,), jnp.float32)
    zi32 = jnp.zeros((16,), jnp.int32)

    def _zrow(i, _):
        for q in range(8):
            rows0[i, pl.ds(q * 16, 16)] = zf32
        return 0
    lax.fori_loop(0, CH, _zrow, 0)

    row0 = s * RPT
    for q in range(4):
        pltpu.sync_copy(rows0, accum_sh.at[pl.ds(row0 + q * CH, CH)])
    pltpu.sync_copy(rows0.at[pl.ds(0, RPT - 4 * CH)],
                    accum_sh.at[pl.ds(row0 + 4 * CH, RPT - 4 * CH)])

    def _zcnt(i, _):
        cnt_local[i, :] = zi32
        return 0
    lax.fori_loop(0, NPAD, _zcnt, 0)
    pltpu.sync_copy(cnt_local.at[pl.ds(0, NPAD // NS)],
                    cntg_sh.at[pl.ds(s * (NPAD // NS), NPAD // NS)])
    plsc.subcore_barrier()

    chunk0 = s * CPT

    def _gather(iref, k):
        pltpu.async_copy(wh_hbm.at[c].at[iref], rows[k], gsem[k])

    def _wait_gather(k):
        pltpu.make_async_copy(wh_hbm.at[c].at[sidx0.at[0]], rows[k],
                              gsem[k]).wait()

    def _scatter(dref, k):
        pltpu.async_copy(rows[k], accum_sh.at[dref], ssem[k], add=True)

    def _wait_scatter(k):
        pltpu.make_async_copy(rows[k], accum_sh.at[didx0.at[0]],
                              ssem[k]).wait()

    def _stage_idx(t, bb, sync):
        base = chunk0 + t * IB
        pltpu.async_copy(e_hbm.at[c, 0, pl.ds(base, IB)], sidx[bb], isem)
        pltpu.async_copy(e_hbm.at[c, 1, pl.ds(base, IB)], didx[bb], isem)
        if sync:
            _wait_idx(bb)

    def _wait_idx(bb):
        pltpu.make_async_copy(e_hbm.at[0, 0, pl.ds(0, IB)], sidx[bb],
                              isem).wait()
        pltpu.make_async_copy(e_hbm.at[0, 1, pl.ds(0, IB)], didx[bb],
                              isem).wait()

    def _hist(dref, jl):
        for q in range(CH // 16):
            idx = dref[jl, pl.ds(q * 16, 16)]
            run, last = plsc.scan_count(idx)
            r = lax.shift_right_logical(idx, 4)
            col = lax.bitwise_and(idx, 15)
            plsc.addupdate_scatter(cnt_local, [r, col], run, mask=last)

    _stage_idx(0, 0, True)
    _gather(sidx0.at[0], 0)
    _gather(sidx0.at[1], 1)

    def _do_block(t, bb):
        @pl.when(t + 1 < NBLK)
        def _():
            _stage_idx(t + 1, 1 - bb, False)

        for jl in range(IB):
            k = jl % 2
            _wait_gather(k)
            _scatter(didx[bb].at[jl], k)
            _hist(didx[bb], jl)
            _wait_scatter(k)
            nxt = jl + 2
            if nxt < IB:
                _gather(sidx[bb].at[nxt], k)
            else:
                if jl == IB - 2:
                    @pl.when(t + 1 < NBLK)
                    def _():
                        _wait_idx(1 - bb)

                @pl.when(t + 1 < NBLK)
                def _():
                    _gather(sidx[1 - bb].at[nxt - IB], k)

    def _pair(u, _):
        _do_block(2 * u, 0)
        _do_block(2 * u + 1, 1)
        return 0

    lax.fori_loop(0, NBLK // 2, _pair, 0)

    @pl.when(s < EXTRA)
    def _():
        base = CPT * NS + s
        pltpu.sync_copy(e_hbm.at[c, 0, pl.ds(base, 1)],
                        sidx0.at[pl.ds(0, 1)])
        pltpu.sync_copy(e_hbm.at[c, 1, pl.ds(base, 1)],
                        didx0.at[pl.ds(0, 1)])
        pltpu.async_copy(wh_hbm.at[c].at[sidx0.at[0]], rows0, g0).wait()
        pltpu.sync_copy(rows0, accum_sh.at[didx0.at[0]], add=True)
        _hist(didx0, 0)

    iota16 = lax.iota(jnp.int32, 16)
    for q in range(NPAD // CH):
        for r in range(8):
            sidx0[q, pl.ds(r * 16, 16)] = q * CH + r * 16 + iota16
    plsc.subcore_barrier()
    for q in range(NPAD // CH):
        pltpu.async_copy(cnt_local.at[pl.ds(q * CH, CH)],
                         cntg_sh.at[sidx0.at[q]], csem, add=True)
        pltpu.make_async_copy(cnt_local.at[pl.ds(q * CH, CH)],
                              cntg_sh.at[sidx0.at[q]], csem).wait()

    plsc.subcore_barrier()

    pltpu.sync_copy(accum_sh.at[pl.ds(row0, RPT)],
                    sums_hbm.at[c, pl.ds(row0, RPT)])
    pltpu.sync_copy(cntg_sh.at[pl.ds(s * (NPAD // NS), NPAD // NS)],
                    cnts_hbm.at[c, pl.ds(s * (NPAD // NS), NPAD // NS)])


def _sc_aggregate(wh, edges):
    mesh = plsc.VectorSubcoreMesh(core_axis_name="c", subcore_axis_name="s")
    k = pl.kernel(
        _sc_agg_body,
        out_type=[
            jax.ShapeDtypeStruct((2, N, D), jnp.float32),
            jax.ShapeDtypeStruct((2, NPAD, 16), jnp.int32),
        ],
        mesh=mesh,
        scratch_types=[
            pltpu.VMEM((IB, CH), jnp.int32),
            pltpu.VMEM((IB, CH), jnp.int32),
            pltpu.VMEM((IB, CH), jnp.int32),
            pltpu.VMEM((IB, CH), jnp.int32),
            pltpu.VMEM((CH, D), jnp.float32),
            pltpu.VMEM((CH, D), jnp.float32),
            pltpu.VMEM((NPAD, 16), jnp.int32),
            pltpu.VMEM_SHARED((N, D), jnp.float32),
            pltpu.VMEM_SHARED((NPAD, 16), jnp.int32),
            pltpu.SemaphoreType.DMA,
            pltpu.SemaphoreType.DMA,
            pltpu.SemaphoreType.DMA,
            pltpu.SemaphoreType.DMA,
            pltpu.SemaphoreType.DMA,
            pltpu.SemaphoreType.DMA,
        ],
        compiler_params=pltpu.CompilerParams(use_tc_tiling_on_sc=False,
                                             needs_layout_passes=False),
    )
    return k(wh, edges)


def _final_body(s0_ref, s1_ref, c0_ref, c1_ref, ft_ref, fc_ref,
                w_ref, b_ref, ot_ref, oc_ref):
    w = w_ref[...]
    b = b_ref[...]
    c1 = jnp.maximum(c1_ref[0].astype(jnp.float32), 1.0)
    h1 = s1_ref[0] / c1
    ot_ref[...] = _leaky(
        jnp.dot(h1, w, preferred_element_type=jnp.float32) + b) + ft_ref[...]
    c0 = jnp.maximum(c0_ref[0].astype(jnp.float32), 1.0)
    h0 = s0_ref[0] / c0
    oc_ref[...] = _leaky(
        jnp.dot(h0, w, preferred_element_type=jnp.float32) + b) + fc_ref[...]


def _tc_final(sums, cnts_col, feat_table, feat_column, W_h, b_h):
    BR = 1000
    grid = (N // BR,)
    out = pl.pallas_call(
        _final_body,
        grid=grid,
        in_specs=[
            pl.BlockSpec((1, BR, D), lambda i: (0, i, 0)),
            pl.BlockSpec((1, BR, D), lambda i: (1, i, 0)),
            pl.BlockSpec((1, BR, 1), lambda i: (0, i, 0)),
            pl.BlockSpec((1, BR, 1), lambda i: (1, i, 0)),
            pl.BlockSpec((BR, D), lambda i: (i, 0)),
            pl.BlockSpec((BR, D), lambda i: (i, 0)),
            pl.BlockSpec((D, D), lambda i: (0, 0)),
            pl.BlockSpec((1, D), lambda i: (0, 0)),
        ],
        out_specs=[
            pl.BlockSpec((BR, D), lambda i: (i, 0)),
            pl.BlockSpec((BR, D), lambda i: (i, 0)),
        ],
        out_shape=[
            jax.ShapeDtypeStruct((N, D), jnp.float32),
            jax.ShapeDtypeStruct((N, D), jnp.float32),
        ],
    )(sums, sums, cnts_col, cnts_col, feat_table, feat_column, W_h, b_h)
    return out


def kernel(feat_table, feat_column, edge_t2c, edge_c2t,
           W_t2c, b_t2c, W_c2t, b_c2t, W_h, b_h):
    Ws = jnp.stack([W_t2c, W_c2t])
    bs = jnp.stack([b_t2c, b_c2t]).reshape(2, 1, D)
    wh = _tc_project(feat_table, feat_column, Ws, bs)

    edges = jnp.stack([edge_t2c.astype(jnp.int32).reshape(2, NCHUNK, CH),
                       edge_c2t.astype(jnp.int32).reshape(2, NCHUNK, CH)])
    sums, cnts = _sc_aggregate(wh, edges)

    cnts_col = cnts.reshape(2, NPAD * 16, 1)[:, :N, :]
    out_table, out_column = _tc_final(sums, cnts_col, feat_table,
                                      feat_column, W_h, b_h.reshape(1, D))
    return out_table, out_column


# speedup vs baseline: 16.4567x; 1.2838x over previous
import jax
import jax.numpy as jnp
from jax import lax
from jax.experimental import pallas as pl
from jax.experimental.pallas import tpu as pltpu
from jax.experimental.pallas import tpu_sc as plsc

N = 10000
E = 320000
D = 128
NS = 16
CH = 128
NCHUNK = E // CH
CPT = NCHUNK // NS
EXTRA = NCHUNK - CPT * NS
IB = 6
NBLK = CPT // IB
RPT = N // NS
NPAD = 640


def _leaky(x):
    return jnp.where(x >= 0, x, 0.01 * x)


def _proj_body(ft_ref, fc_ref, w_ref, b_ref, o_ref):
    e = pl.program_id(0)
    x = jnp.where(e == 0, ft_ref[...], fc_ref[...])
    acc = jnp.dot(x, w_ref[0], preferred_element_type=jnp.float32)
    o_ref[0] = _leaky(acc + b_ref[0]).astype(jnp.bfloat16)


def _tc_project(feat_table, feat_column, Ws, bs):
    BR = 1000
    grid = (2, N // BR)
    return pl.pallas_call(
        _proj_body,
        grid=grid,
        in_specs=[
            pl.BlockSpec((BR, D), lambda e, i: (i, 0)),
            pl.BlockSpec((BR, D), lambda e, i: (i, 0)),
            pl.BlockSpec((1, D, D), lambda e, i: (e, 0, 0)),
            pl.BlockSpec((1, 1, D), lambda e, i: (e, 0, 0)),
        ],
        out_specs=pl.BlockSpec((1, BR, D), lambda e, i: (e, i, 0)),
        out_shape=jax.ShapeDtypeStruct((2, N, D), jnp.bfloat16),
    )(feat_table, feat_column, Ws, bs)


def _sc_agg_body(wh_hbm, e_hbm, sums_hbm, cnts_hbm,
                 sidx0, sidx1, didx0, didx1,
                 rows0, rows1, rows2, rows3, cnt_local,
                 accum_sh, cntg_sh,
                 g0, g1, g2, g3, s0, s1, s2, s3, isem, csem):
    c = lax.axis_index("c")
    s = lax.axis_index("s")
    rows = (rows0, rows1, rows2, rows3)
    gsem = (g0, g1, g2, g3)
    ssem = (s0, s1, s2, s3)
    sidx = (sidx0, sidx1)
    didx = (didx0, didx1)

    zb16 = jnp.zeros((32,), jnp.bfloat16)
    zi32 = jnp.zeros((16,), jnp.int32)

    def _zrow(i, _):
        for q in range(4):
            rows0[i, pl.ds(q * 32, 32)] = zb16
        return 0
    lax.fori_loop(0, CH, _zrow, 0)

    row0 = s * RPT
    for q in range(4):
        pltpu.sync_copy(rows0, accum_sh.at[pl.ds(row0 + q * CH, CH)])
    pltpu.sync_copy(rows0.at[pl.ds(0, RPT - 4 * CH)],
                    accum_sh.at[pl.ds(row0 + 4 * CH, RPT - 4 * CH)])

    def _zcnt(i, _):
        cnt_local[i, :] = zi32
        return 0
    lax.fori_loop(0, NPAD, _zcnt, 0)
    pltpu.sync_copy(cnt_local.at[pl.ds(0, NPAD // NS)],
                    cntg_sh.at[pl.ds(s * (NPAD // NS), NPAD // NS)])
    plsc.subcore_barrier()

    chunk0 = s * CPT

    def _gather(iref, k):
        pltpu.async_copy(wh_hbm.at[c].at[iref], rows[k], gsem[k])

    def _wait_gather(k):
        pltpu.make_async_copy(wh_hbm.at[c].at[sidx0.at[0]], rows[k],
                              gsem[k]).wait()

    def _scatter(dref, k):
        pltpu.async_copy(rows[k], accum_sh.at[dref], ssem[k], add=True)

    def _wait_scatter(k):
        pltpu.make_async_copy(rows[k], accum_sh.at[didx0.at[0]],
                              ssem[k]).wait()

    def _stage_idx(t, bb, sync):
        base = chunk0 + t * IB
        pltpu.async_copy(e_hbm.at[c, 0, pl.ds(base, IB)], sidx[bb], isem)
        pltpu.async_copy(e_hbm.at[c, 1, pl.ds(base, IB)], didx[bb], isem)
        if sync:
            _wait_idx(bb)

    def _wait_idx(bb):
        pltpu.make_async_copy(e_hbm.at[0, 0, pl.ds(0, IB)], sidx[bb],
                              isem).wait()
        pltpu.make_async_copy(e_hbm.at[0, 1, pl.ds(0, IB)], didx[bb],
                              isem).wait()

    def _hist(dref, jl):
        for q in range(CH // 16):
            idx = dref[jl, pl.ds(q * 16, 16)]
            run, last = plsc.scan_count(idx)
            r = lax.shift_right_logical(idx, 4)
            col = lax.bitwise_and(idx, 15)
            plsc.addupdate_scatter(cnt_local, [r, col], run, mask=last)

    _stage_idx(0, 0, True)
    for k in range(4):
        _gather(sidx0.at[k], k)

    def _do_block(t, bb):
        @pl.when(t + 1 < NBLK)
        def _():
            _stage_idx(t + 1, 1 - bb, False)

        for jl in range(IB):
            k = (2 * bb + jl) % 4
            _wait_gather(k)
            _scatter(didx[bb].at[jl], k)
            _hist(didx[bb], jl)
            _wait_scatter(k)
            nxt = jl + 4
            if jl == IB - 4:
                @pl.when(t + 1 < NBLK)
                def _():
                    _wait_idx(1 - bb)
            if nxt < IB:
                _gather(sidx[bb].at[nxt], k)
            else:
                @pl.when(t + 1 < NBLK)
                def _():
                    _gather(sidx[1 - bb].at[nxt - IB], k)

    def _pair(u, _):
        _do_block(2 * u, 0)
        _do_block(2 * u + 1, 1)
        return 0

    lax.fori_loop(0, NBLK // 2, _pair, 0)

    @pl.when(s < EXTRA)
    def _():
        base = CPT * NS + s
        pltpu.sync_copy(e_hbm.at[c, 0, pl.ds(base, 1)],
                        sidx0.at[pl.ds(0, 1)])
        pltpu.sync_copy(e_hbm.at[c, 1, pl.ds(base, 1)],
                        didx0.at[pl.ds(0, 1)])
        pltpu.async_copy(wh_hbm.at[c].at[sidx0.at[0]], rows0, g0).wait()
        pltpu.sync_copy(rows0, accum_sh.at[didx0.at[0]], add=True)
        _hist(didx0, 0)

    iota16 = lax.iota(jnp.int32, 16)
    for q in range(NPAD // CH):
        for r in range(8):
            sidx0[q, pl.ds(r * 16, 16)] = q * CH + r * 16 + iota16
    plsc.subcore_barrier()
    for q in range(NPAD // CH):
        pltpu.async_copy(cnt_local.at[pl.ds(q * CH, CH)],
                         cntg_sh.at[sidx0.at[q]], csem, add=True)
        pltpu.make_async_copy(cnt_local.at[pl.ds(q * CH, CH)],
                              cntg_sh.at[sidx0.at[q]], csem).wait()

    plsc.subcore_barrier()

    pltpu.sync_copy(accum_sh.at[pl.ds(row0, RPT)],
                    sums_hbm.at[c, pl.ds(row0, RPT)])
    pltpu.sync_copy(cntg_sh.at[pl.ds(s * (NPAD // NS), NPAD // NS)],
                    cnts_hbm.at[c, pl.ds(s * (NPAD // NS), NPAD // NS)])


def _sc_aggregate(wh, edges):
    mesh = plsc.VectorSubcoreMesh(core_axis_name="c", subcore_axis_name="s")
    k = pl.kernel(
        _sc_agg_body,
        out_type=[
            jax.ShapeDtypeStruct((2, N, D), jnp.bfloat16),
            jax.ShapeDtypeStruct((2, NPAD, 16), jnp.int32),
        ],
        mesh=mesh,
        scratch_types=[
            pltpu.VMEM((IB, CH), jnp.int32),
            pltpu.VMEM((IB, CH), jnp.int32),
            pltpu.VMEM((IB, CH), jnp.int32),
            pltpu.VMEM((IB, CH), jnp.int32),
            pltpu.VMEM((CH, D), jnp.bfloat16),
            pltpu.VMEM((CH, D), jnp.bfloat16),
            pltpu.VMEM((CH, D), jnp.bfloat16),
            pltpu.VMEM((CH, D), jnp.bfloat16),
            pltpu.VMEM((NPAD, 16), jnp.int32),
            pltpu.VMEM_SHARED((N, D), jnp.bfloat16),
            pltpu.VMEM_SHARED((NPAD, 16), jnp.int32),
            pltpu.SemaphoreType.DMA,
            pltpu.SemaphoreType.DMA,
            pltpu.SemaphoreType.DMA,
            pltpu.SemaphoreType.DMA,
            pltpu.SemaphoreType.DMA,
            pltpu.SemaphoreType.DMA,
            pltpu.SemaphoreType.DMA,
            pltpu.SemaphoreType.DMA,
            pltpu.SemaphoreType.DMA,
            pltpu.SemaphoreType.DMA,
        ],
        compiler_params=pltpu.CompilerParams(use_tc_tiling_on_sc=False,
                                             needs_layout_passes=False),
    )
    return k(wh, edges)


def _final_body(s0_ref, s1_ref, c0_ref, c1_ref, ft_ref, fc_ref,
                w_ref, b_ref, ot_ref, oc_ref):
    w = w_ref[...]
    b = b_ref[...]
    c1 = jnp.maximum(c1_ref[0].astype(jnp.float32), 1.0)
    h1 = s1_ref[0].astype(jnp.float32) / c1
    ot_ref[...] = _leaky(
        jnp.dot(h1, w, preferred_element_type=jnp.float32) + b) + ft_ref[...]
    c0 = jnp.maximum(c0_ref[0].astype(jnp.float32), 1.0)
    h0 = s0_ref[0].astype(jnp.float32) / c0
    oc_ref[...] = _leaky(
        jnp.dot(h0, w, preferred_element_type=jnp.float32) + b) + fc_ref[...]


def _tc_final(sums, cnts_col, feat_table, feat_column, W_h, b_h):
    BR = 1000
    grid = (N // BR,)
    out = pl.pallas_call(
        _final_body,
        grid=grid,
        in_specs=[
            pl.BlockSpec((1, BR, D), lambda i: (0, i, 0)),
            pl.BlockSpec((1, BR, D), lambda i: (1, i, 0)),
            pl.BlockSpec((1, BR, 1), lambda i: (0, i, 0)),
            pl.BlockSpec((1, BR, 1), lambda i: (1, i, 0)),
            pl.BlockSpec((BR, D), lambda i: (i, 0)),
            pl.BlockSpec((BR, D), lambda i: (i, 0)),
            pl.BlockSpec((D, D), lambda i: (0, 0)),
            pl.BlockSpec((1, D), lambda i: (0, 0)),
        ],
        out_specs=[
            pl.BlockSpec((BR, D), lambda i: (i, 0)),
            pl.BlockSpec((BR, D), lambda i: (i, 0)),
        ],
        out_shape=[
            jax.ShapeDtypeStruct((N, D), jnp.float32),
            jax.ShapeDtypeStruct((N, D), jnp.float32),
        ],
    )(sums, sums, cnts_col, cnts_col, feat_table, feat_column, W_h, b_h)
    return out


def kernel(feat_table, feat_column, edge_t2c, edge_c2t,
           W_t2c, b_t2c, W_c2t, b_c2t, W_h, b_h):
    Ws = jnp.stack([W_t2c, W_c2t])
    bs = jnp.stack([b_t2c, b_c2t]).reshape(2, 1, D)
    wh = _tc_project(feat_table, feat_column, Ws, bs)

    edges = jnp.stack([edge_t2c.astype(jnp.int32).reshape(2, NCHUNK, CH),
                       edge_c2t.astype(jnp.int32).reshape(2, NCHUNK, CH)])
    sums, cnts = _sc_aggregate(wh, edges)

    cnts_col = cnts.reshape(2, NPAD * 16, 1)[:, :N, :]
    out_table, out_column = _tc_final(sums, cnts_col, feat_table,
                                      feat_column, W_h, b_h.reshape(1, D))
    return out_table, out_column
